# baseline (speedup 1.0000x reference)
import jax
import jax.numpy as jnp
from jax.experimental import pallas as pl
from jax.experimental.pallas import tpu as pltpu

_PACK = 4
_WIN = 24
_FEATS = 4
_ROW_TILE = 512


def _tile_kernel(zp_ref, w_ref, b_ref, o_ref):
    h = jnp.dot(zp_ref[...].astype(jnp.bfloat16), w_ref[...],
                preferred_element_type=jnp.float32)
    h = (h + b_ref[...]) * 0.5
    o_ref[...] = jnp.tanh(h) * 0.5 + 0.5


def kernel(z, W_bd, b_t):
    B, latent = z.shape
    kp, cols = W_bd.shape
    n = cols // _PACK

    rows_raw = -(-B // _PACK)
    rows = -(-rows_raw // _ROW_TILE) * _ROW_TILE
    B_pad = rows * _PACK
    if B_pad != B:
        z = jnp.pad(z, ((0, B_pad - B), (0, 0)))
    zp = z.reshape(rows, _PACK * latent)

    wb = W_bd.astype(jnp.bfloat16)

    out = pl.pallas_call(
        _tile_kernel,
        out_shape=jax.ShapeDtypeStruct((rows, cols), jnp.float32),
        grid=(rows // _ROW_TILE,),
        in_specs=[
            pl.BlockSpec((_ROW_TILE, kp), lambda i: (i, 0)),
            pl.BlockSpec((kp, cols), lambda i: (0, 0)),
            pl.BlockSpec((1, cols), lambda i: (0, 0)),
        ],
        out_specs=pl.BlockSpec((_ROW_TILE, cols), lambda i: (i, 0)),
        compiler_params=pltpu.CompilerParams(
            dimension_semantics=("parallel",)),
    )(zp, wb, b_t)

    flat = out.reshape(B_pad, n)
    if B_pad != B:
        flat = flat[:B]
    return flat.reshape(B, _WIN, _FEATS)


# speedup vs baseline: 19.7361x; 19.7361x over previous
import jax
import jax.numpy as jnp
from jax.experimental import pallas as pl
from jax.experimental.pallas import tpu as pltpu

_WIN = 24
_FEATS = 4
_N = _WIN * _FEATS
_LANE_TILE = 2048


def _gen_kernel(z_ref, wt_ref, bt_ref, o_ref):
    ht = jax.lax.dot_general(
        wt_ref[...], z_ref[...].astype(jnp.bfloat16),
        (((1,), (1,)), ((), ())),
        preferred_element_type=jnp.float32,
    )
    ht = (ht + bt_ref[...]) * 0.5
    o_ref[...] = jnp.tanh(ht) * 0.5 + 0.5


def kernel(z, W_bd, b_t):
    B, latent = z.shape

    bp = -(-B // _LANE_TILE) * _LANE_TILE
    if bp != B:
        z = jnp.pad(z, ((0, bp - B), (0, 0)))

    wt = W_bd[:latent, :_N].T.astype(jnp.bfloat16)
    bt = b_t[:, :_N].reshape(_N, 1)

    ht = pl.pallas_call(
        _gen_kernel,
        out_shape=jax.ShapeDtypeStruct((_N, bp), jnp.float32),
        grid=(bp // _LANE_TILE,),
        in_specs=[
            pl.BlockSpec((_LANE_TILE, latent), lambda i: (i, 0)),
            pl.BlockSpec((_N, latent), lambda i: (0, 0)),
            pl.BlockSpec((_N, 1), lambda i: (0, 0)),
        ],
        out_specs=pl.BlockSpec((_N, _LANE_TILE), lambda i: (0, i)),
        compiler_params=pltpu.CompilerParams(
            dimension_semantics=("parallel",)),
    )(z, wt, bt)

    if bp != B:
        ht = ht[:, :B]
    return ht.reshape(_WIN, _FEATS, B).transpose(2, 0, 1)
